# Initial kernel scaffold; baseline (speedup 1.0000x reference)
import functools

import jax
import jax.numpy as jnp
from jax import lax
from jax.experimental import pallas as pl
from jax.experimental.pallas import tpu as pltpu
from jax.experimental.pallas import tpu_sc as plsc

N = 10000
E = 320000
D = 128

NC = 2
NS = 16
LANES = 16
NW = NC * NS

N_PAD = 10240
ROWS_S = N_PAD // NS
E_PAD = 327680
EW = E_PAD // NW
CHUNK = 128
NCHUNK = EW // CHUNK
ZROWS = 160

_mesh = plsc.VectorSubcoreMesh(core_axis_name="c", subcore_axis_name="s",
                               num_cores=NC, num_subcores=NS)


@functools.partial(
    pl.kernel,
    out_type=jax.ShapeDtypeStruct((NC, N_PAD), jnp.int32),
    mesh=_mesh,
    scratch_types=[
        pltpu.VMEM((N_PAD,), jnp.int32),
        pltpu.VMEM((EW,), jnp.int32),
        pltpu.VMEM((NS, ROWS_S), jnp.int32),
        pltpu.VMEM((ROWS_S,), jnp.int32),
        pltpu.VMEM_SHARED((NS, N_PAD), jnp.int32),
    ],
)
def _deg_kernel(dst_hbm, out_hbm, hist, dstbuf, redbuf, resbuf, stage):
    c = lax.axis_index("c")
    s = lax.axis_index("s")
    wid = s * NC + c

    zeros16 = jnp.zeros((LANES,), jnp.int32)
    ones16 = jnp.ones((LANES,), jnp.int32)

    def zero_body(i, carry):
        hist[pl.ds(i * LANES, LANES)] = zeros16
        return carry

    lax.fori_loop(0, N_PAD // LANES, zero_body, 0)

    pltpu.sync_copy(dst_hbm.at[pl.ds(wid * EW, EW)], dstbuf)

    def hist_body(i, carry):
        idx = dstbuf[pl.ds(i * LANES, LANES)]
        plsc.addupdate_scatter(hist, [idx], ones16)
        return carry

    lax.fori_loop(0, EW // LANES, hist_body, 0)

    pltpu.sync_copy(hist, stage.at[s])
    plsc.subcore_barrier()

    def fetch_body(i, carry):
        pltpu.sync_copy(stage.at[i, pl.ds(s * ROWS_S, ROWS_S)], redbuf.at[i])
        return carry

    lax.fori_loop(0, NS, fetch_body, 0)

    def reduce_body(i, carry):
        v = redbuf[0, pl.ds(i * LANES, LANES)]
        for j in range(1, NS):
            v = v + redbuf[j, pl.ds(i * LANES, LANES)]
        resbuf[pl.ds(i * LANES, LANES)] = v
        return carry

    lax.fori_loop(0, ROWS_S // LANES, reduce_body, 0)

    pltpu.sync_copy(resbuf, out_hbm.at[c, pl.ds(s * ROWS_S, ROWS_S)])


def _prescale_body(hist_ref, x_ref, o_ref):
    deg = (hist_ref[..., 0] + hist_ref[..., 1] + 1).astype(jnp.float32)
    dinv = lax.rsqrt(deg)
    o_ref[...] = x_ref[...] * dinv[:, None]


_B2 = 1024


def _prescale(hist_t, x_pad):
    return pl.pallas_call(
        _prescale_body,
        grid=(N_PAD // _B2,),
        in_specs=[
            pl.BlockSpec((_B2, 2), lambda i: (i, 0)),
            pl.BlockSpec((_B2, D), lambda i: (i, 0)),
        ],
        out_specs=pl.BlockSpec((_B2, D), lambda i: (i, 0)),
        out_shape=jax.ShapeDtypeStruct((N_PAD, D), jnp.float32),
    )(hist_t, x_pad)


@functools.partial(
    pl.kernel,
    out_type=jax.ShapeDtypeStruct((NC, N_PAD, D), jnp.float32),
    mesh=_mesh,
    scratch_types=[
        pltpu.VMEM((NCHUNK, CHUNK), jnp.int32),
        pltpu.VMEM((NCHUNK, CHUNK), jnp.int32),
        pltpu.VMEM((CHUNK, D), jnp.float32),
        pltpu.VMEM((ZROWS, D), jnp.float32),
        pltpu.VMEM_SHARED((N_PAD, D), jnp.float32),
        pltpu.SemaphoreType.DMA,
    ],
)
def _agg_kernel(x2_hbm, src_hbm, dst_hbm, out_hbm,
                sidx, didx, rows, zbuf, acc_s, sem):
    c = lax.axis_index("c")
    s = lax.axis_index("s")
    wid = s * NC + c

    zeros16 = jnp.zeros((LANES,), jnp.float32)

    def zero_body(i, carry):
        zbuf[i // (D // LANES), pl.ds((i % (D // LANES)) * LANES, LANES)] = zeros16
        return carry

    lax.fori_loop(0, ZROWS * (D // LANES), zero_body, 0)

    base_row = s * ROWS_S
    for t in range(ROWS_S // ZROWS):
        pltpu.sync_copy(zbuf, acc_s.at[pl.ds(base_row + t * ZROWS, ZROWS), :])
    plsc.subcore_barrier()

    pltpu.sync_copy(src_hbm.at[pl.ds(wid * NCHUNK, NCHUNK), :], sidx)
    pltpu.sync_copy(dst_hbm.at[pl.ds(wid * NCHUNK, NCHUNK), :], didx)

    def edge_body(j, carry):
        pltpu.async_copy(x2_hbm.at[sidx.at[j]], rows, sem).wait()
        pltpu.sync_copy(rows, acc_s.at[didx.at[j]], add=True)
        return carry

    lax.fori_loop(0, NCHUNK, edge_body, 0)
    plsc.subcore_barrier()

    for t in range(ROWS_S // ZROWS):
        r0 = base_row + t * ZROWS
        pltpu.sync_copy(acc_s.at[pl.ds(r0, ZROWS), :], zbuf)
        pltpu.sync_copy(zbuf, out_hbm.at[c, pl.ds(r0, ZROWS), :])


def _out_body(hist_ref, a0_ref, a1_ref, x2_ref, w_ref, b_ref, a_ref, o_ref):
    deg = (hist_ref[..., 0] + hist_ref[..., 1] + 1).astype(jnp.float32)
    dinv = lax.rsqrt(deg)
    agg = (a0_ref[...] + a1_ref[...] + x2_ref[...]) * dinv[:, None]
    h = jnp.dot(agg, w_ref[...], preferred_element_type=jnp.float32)
    h = h + b_ref[...]
    o_ref[...] = jnp.where(h > 0, h, a_ref[...] * h)


_B4 = 512


def _finalize(hist_t, acc0, acc1, x2, W, b2, a2):
    return pl.pallas_call(
        _out_body,
        grid=(N_PAD // _B4,),
        in_specs=[
            pl.BlockSpec((_B4, 2), lambda i: (i, 0)),
            pl.BlockSpec((_B4, D), lambda i: (i, 0)),
            pl.BlockSpec((_B4, D), lambda i: (i, 0)),
            pl.BlockSpec((_B4, D), lambda i: (i, 0)),
            pl.BlockSpec((D, D), lambda i: (0, 0)),
            pl.BlockSpec((1, D), lambda i: (0, 0)),
            pl.BlockSpec((1, D), lambda i: (0, 0)),
        ],
        out_specs=pl.BlockSpec((_B4, D), lambda i: (i, 0)),
        out_shape=jax.ShapeDtypeStruct((N_PAD, D), jnp.float32),
    )(hist_t, acc0, acc1, x2, W, b2, a2)


def kernel(x, edge_index, edge_type, W, b, a):
    del edge_type
    src = edge_index[0].astype(jnp.int32)
    dst = edge_index[1].astype(jnp.int32)
    src_p = jnp.concatenate([src, jnp.full((E_PAD - E,), N, jnp.int32)])
    dst_p = jnp.concatenate([dst, jnp.full((E_PAD - E,), N_PAD - 1, jnp.int32)])
    src2d = src_p.reshape(NW * NCHUNK, CHUNK)
    dst2d = dst_p.reshape(NW * NCHUNK, CHUNK)
    x_pad = jnp.zeros((N_PAD, D), jnp.float32).at[:N].set(x)

    hist = _deg_kernel(dst_p)
    hist_t = hist.T
    x2 = _prescale(hist_t, x_pad)
    acc = _agg_kernel(x2, src2d, dst2d)
    out = _finalize(hist_t, acc[0], acc[1], x2, W,
                    b.reshape(1, D), a.reshape(1, D))
    return out[:N]


# speedup vs baseline: 10.8504x; 10.8504x over previous
import functools

import jax
import jax.numpy as jnp
from jax import lax
from jax.experimental import pallas as pl
from jax.experimental.pallas import tpu as pltpu
from jax.experimental.pallas import tpu_sc as plsc

N = 10000
E = 320000
D = 128

NC = 2
NS = 16
LANES = 16
NW = NC * NS

N_PAD = 10240
ROWS_S = N_PAD // NS
E_PAD = 327680
EW = E_PAD // NW
CHUNK = 128
NCHUNK = EW // CHUNK
ZROWS = 64

_mesh = plsc.VectorSubcoreMesh(core_axis_name="c", subcore_axis_name="s",
                               num_cores=NC, num_subcores=NS)


@functools.partial(
    pl.kernel,
    out_type=jax.ShapeDtypeStruct((NC, N_PAD), jnp.int32),
    mesh=_mesh,
    scratch_types=[
        pltpu.VMEM((N_PAD,), jnp.int32),
        pltpu.VMEM((EW,), jnp.int32),
        pltpu.VMEM((NS, ROWS_S), jnp.int32),
        pltpu.VMEM((ROWS_S,), jnp.int32),
        pltpu.VMEM_SHARED((NS, N_PAD), jnp.int32),
    ],
    compiler_params=pltpu.CompilerParams(needs_layout_passes=False),
)
def _deg_kernel(dst_hbm, out_hbm, hist, dstbuf, redbuf, resbuf, stage):
    c = lax.axis_index("c")
    s = lax.axis_index("s")
    wid = s * NC + c

    zeros16 = jnp.zeros((LANES,), jnp.int32)
    ones16 = jnp.ones((LANES,), jnp.int32)

    def zero_body(i, carry):
        hist[pl.ds(i * LANES, LANES)] = zeros16
        return carry

    lax.fori_loop(0, N_PAD // LANES, zero_body, 0)

    pltpu.sync_copy(dst_hbm.at[pl.ds(wid * EW, EW)], dstbuf)

    def hist_body(i, carry):
        idx = dstbuf[pl.ds(i * LANES, LANES)]
        plsc.addupdate_scatter(hist, [idx], ones16)
        return carry

    lax.fori_loop(0, EW // LANES, hist_body, 0)

    pltpu.sync_copy(hist, stage.at[s])
    plsc.subcore_barrier()

    def fetch_body(i, carry):
        pltpu.sync_copy(stage.at[i, pl.ds(s * ROWS_S, ROWS_S)], redbuf.at[i])
        return carry

    lax.fori_loop(0, NS, fetch_body, 0)

    def reduce_body(i, carry):
        v = redbuf[0, pl.ds(i * LANES, LANES)]
        for j in range(1, NS):
            v = v + redbuf[j, pl.ds(i * LANES, LANES)]
        resbuf[pl.ds(i * LANES, LANES)] = v
        return carry

    lax.fori_loop(0, ROWS_S // LANES, reduce_body, 0)

    pltpu.sync_copy(resbuf, out_hbm.at[c, pl.ds(s * ROWS_S, ROWS_S)])


def _prescale_body(hist_ref, x_ref, o_ref):
    deg = (hist_ref[..., 0] + hist_ref[..., 1] + 1).astype(jnp.float32)
    dinv = lax.rsqrt(deg)
    o_ref[...] = x_ref[...] * dinv[:, None]


_B2 = 1024


def _prescale(hist_t, x_pad):
    return pl.pallas_call(
        _prescale_body,
        grid=(N_PAD // _B2,),
        in_specs=[
            pl.BlockSpec((_B2, 2), lambda i: (i, 0)),
            pl.BlockSpec((_B2, D), lambda i: (i, 0)),
        ],
        out_specs=pl.BlockSpec((_B2, D), lambda i: (i, 0)),
        out_shape=jax.ShapeDtypeStruct((N_PAD, D), jnp.float32),
    )(hist_t, x_pad)


@functools.partial(
    pl.kernel,
    out_type=jax.ShapeDtypeStruct((NC, N_PAD, D), jnp.float32),
    mesh=_mesh,
    scratch_types=[
        pltpu.VMEM((NCHUNK, CHUNK), jnp.int32),
        pltpu.VMEM((NCHUNK, CHUNK), jnp.int32),
        pltpu.VMEM((CHUNK, D), jnp.float32),
        pltpu.VMEM((ZROWS, D), jnp.float32),
        pltpu.VMEM_SHARED((N_PAD, D), jnp.float32),
        pltpu.SemaphoreType.DMA,
    ],
)
def _agg_kernel(x2_hbm, src_hbm, dst_hbm, out_hbm,
                sidx, didx, rows, zbuf, acc_s, sem):
    c = lax.axis_index("c")
    s = lax.axis_index("s")
    wid = s * NC + c

    zeros16 = jnp.zeros((LANES,), jnp.float32)

    def zero_body(i, carry):
        zbuf[i // (D // LANES), pl.ds((i % (D // LANES)) * LANES, LANES)] = zeros16
        return carry

    lax.fori_loop(0, ZROWS * (D // LANES), zero_body, 0)

    base_row = s * ROWS_S
    for t in range(ROWS_S // ZROWS):
        pltpu.sync_copy(zbuf, acc_s.at[pl.ds(base_row + t * ZROWS, ZROWS), :])
    plsc.subcore_barrier()

    pltpu.sync_copy(src_hbm.at[pl.ds(wid * NCHUNK, NCHUNK), :], sidx)
    pltpu.sync_copy(dst_hbm.at[pl.ds(wid * NCHUNK, NCHUNK), :], didx)

    def edge_body(j, carry):
        pltpu.async_copy(x2_hbm.at[sidx.at[j]], rows, sem).wait()
        pltpu.sync_copy(rows, acc_s.at[didx.at[j]], add=True)
        return carry

    lax.fori_loop(0, NCHUNK, edge_body, 0)
    plsc.subcore_barrier()

    for t in range(ROWS_S // ZROWS):
        r0 = base_row + t * ZROWS
        pltpu.sync_copy(acc_s.at[pl.ds(r0, ZROWS), :], zbuf)
        pltpu.sync_copy(zbuf, out_hbm.at[c, pl.ds(r0, ZROWS), :])


def _out_body(hist_ref, a0_ref, a1_ref, x2_ref, w_ref, b_ref, a_ref, o_ref):
    deg = (hist_ref[..., 0] + hist_ref[..., 1] + 1).astype(jnp.float32)
    dinv = lax.rsqrt(deg)
    agg = (a0_ref[...] + a1_ref[...] + x2_ref[...]) * dinv[:, None]
    h = jnp.dot(agg, w_ref[...], preferred_element_type=jnp.float32)
    h = h + b_ref[...]
    o_ref[...] = jnp.where(h > 0, h, a_ref[...] * h)


_B4 = 512


def _finalize(hist_t, acc0, acc1, x2, W, b2, a2):
    return pl.pallas_call(
        _out_body,
        grid=(N_PAD // _B4,),
        in_specs=[
            pl.BlockSpec((_B4, 2), lambda i: (i, 0)),
            pl.BlockSpec((_B4, D), lambda i: (i, 0)),
            pl.BlockSpec((_B4, D), lambda i: (i, 0)),
            pl.BlockSpec((_B4, D), lambda i: (i, 0)),
            pl.BlockSpec((D, D), lambda i: (0, 0)),
            pl.BlockSpec((1, D), lambda i: (0, 0)),
            pl.BlockSpec((1, D), lambda i: (0, 0)),
        ],
        out_specs=pl.BlockSpec((_B4, D), lambda i: (i, 0)),
        out_shape=jax.ShapeDtypeStruct((N_PAD, D), jnp.float32),
    )(hist_t, acc0, acc1, x2, W, b2, a2)


def kernel(x, edge_index, edge_type, W, b, a):
    del edge_type
    src = edge_index[0].astype(jnp.int32)
    dst = edge_index[1].astype(jnp.int32)
    src_p = jnp.concatenate([src, jnp.full((E_PAD - E,), N, jnp.int32)])
    dst_p = jnp.concatenate([dst, jnp.full((E_PAD - E,), N_PAD - 1, jnp.int32)])
    src2d = src_p.reshape(NW * NCHUNK, CHUNK)
    dst2d = dst_p.reshape(NW * NCHUNK, CHUNK)
    x_pad = jnp.zeros((N_PAD, D), jnp.float32).at[:N].set(x)

    hist = _deg_kernel(dst_p)
    hist_t = hist.T
    x2 = _prescale(hist_t, x_pad)
    acc = _agg_kernel(x2, src2d, dst2d)
    out = _finalize(hist_t, acc[0], acc[1], x2, W,
                    b.reshape(1, D), a.reshape(1, D))
    return out[:N]


# speedup vs baseline: 11.8694x; 1.0939x over previous
import functools

import jax
import jax.numpy as jnp
from jax import lax
from jax.experimental import pallas as pl
from jax.experimental.pallas import tpu as pltpu
from jax.experimental.pallas import tpu_sc as plsc

N = 10000
E = 320000
D = 128

NC = 2
NS = 16
LANES = 16
NW = NC * NS

N_PAD = 10240
ROWS_S = N_PAD // NS
E_PAD = 327680
EW = E_PAD // NW
CHUNK = 128
NCHUNK = EW // CHUNK
ZROWS = 32
NB = NCHUNK // 2

_mesh = plsc.VectorSubcoreMesh(core_axis_name="c", subcore_axis_name="s",
                               num_cores=NC, num_subcores=NS)


@functools.partial(
    pl.kernel,
    out_type=jax.ShapeDtypeStruct((NC, N_PAD), jnp.int32),
    mesh=_mesh,
    scratch_types=[
        pltpu.VMEM((N_PAD,), jnp.int32),
        pltpu.VMEM((EW,), jnp.int32),
        pltpu.VMEM((NS, ROWS_S), jnp.int32),
        pltpu.VMEM((ROWS_S,), jnp.int32),
        pltpu.VMEM_SHARED((NS, N_PAD), jnp.int32),
    ],
    compiler_params=pltpu.CompilerParams(needs_layout_passes=False),
)
def _deg_kernel(dst_hbm, out_hbm, hist, dstbuf, redbuf, resbuf, stage):
    c = lax.axis_index("c")
    s = lax.axis_index("s")
    wid = s * NC + c

    zeros16 = jnp.zeros((LANES,), jnp.int32)
    ones16 = jnp.ones((LANES,), jnp.int32)

    def zero_body(i, carry):
        hist[pl.ds(i * LANES, LANES)] = zeros16
        return carry

    lax.fori_loop(0, N_PAD // LANES, zero_body, 0)

    pltpu.sync_copy(dst_hbm.at[pl.ds(wid * EW, EW)], dstbuf)

    def hist_body(i, carry):
        idx = dstbuf[pl.ds(i * LANES, LANES)]
        plsc.addupdate_scatter(hist, [idx], ones16)
        return carry

    lax.fori_loop(0, EW // LANES, hist_body, 0)

    pltpu.sync_copy(hist, stage.at[s])
    plsc.subcore_barrier()

    def fetch_body(i, carry):
        pltpu.sync_copy(stage.at[i, pl.ds(s * ROWS_S, ROWS_S)], redbuf.at[i])
        return carry

    lax.fori_loop(0, NS, fetch_body, 0)

    def reduce_body(i, carry):
        v = redbuf[0, pl.ds(i * LANES, LANES)]
        for j in range(1, NS):
            v = v + redbuf[j, pl.ds(i * LANES, LANES)]
        resbuf[pl.ds(i * LANES, LANES)] = v
        return carry

    lax.fori_loop(0, ROWS_S // LANES, reduce_body, 0)

    pltpu.sync_copy(resbuf, out_hbm.at[c, pl.ds(s * ROWS_S, ROWS_S)])


def _prescale_body(hist_ref, x_ref, o_ref):
    deg = (hist_ref[..., 0] + hist_ref[..., 1] + 1).astype(jnp.float32)
    dinv = lax.rsqrt(deg)
    o_ref[...] = x_ref[...] * dinv[:, None]


_B2 = 1024


def _prescale(hist_t, x_pad):
    return pl.pallas_call(
        _prescale_body,
        grid=(N_PAD // _B2,),
        in_specs=[
            pl.BlockSpec((_B2, 2), lambda i: (i, 0)),
            pl.BlockSpec((_B2, D), lambda i: (i, 0)),
        ],
        out_specs=pl.BlockSpec((_B2, D), lambda i: (i, 0)),
        out_shape=jax.ShapeDtypeStruct((N_PAD, D), jnp.float32),
    )(hist_t, x_pad)


@functools.partial(
    pl.kernel,
    out_type=jax.ShapeDtypeStruct((NC, N_PAD, D), jnp.float32),
    mesh=_mesh,
    scratch_types=[
        pltpu.VMEM((NB, CHUNK), jnp.int32),
        pltpu.VMEM((NB, CHUNK), jnp.int32),
        pltpu.VMEM((CHUNK, D), jnp.float32),
        pltpu.VMEM((CHUNK, D), jnp.float32),
        pltpu.VMEM((ZROWS, D), jnp.float32),
        pltpu.VMEM_SHARED((N_PAD, D), jnp.float32),
        pltpu.SemaphoreType.DMA,
        pltpu.SemaphoreType.DMA,
    ],
)
def _agg_kernel(x2_hbm, src_hbm, dst_hbm, out_hbm,
                sidx, didx, rows0, rows1, zbuf, acc_s, sem0, sem1):
    c = lax.axis_index("c")
    s = lax.axis_index("s")
    wid = s * NC + c

    zeros16 = jnp.zeros((LANES,), jnp.float32)

    def zero_body(i, carry):
        zbuf[i // (D // LANES), pl.ds((i % (D // LANES)) * LANES, LANES)] = zeros16
        return carry

    lax.fori_loop(0, ZROWS * (D // LANES), zero_body, 0)

    base_row = s * ROWS_S
    for t in range(ROWS_S // ZROWS):
        pltpu.sync_copy(zbuf, acc_s.at[pl.ds(base_row + t * ZROWS, ZROWS), :])
    plsc.subcore_barrier()

    for half in range(2):
        cbase = wid * NCHUNK + half * NB
        pltpu.sync_copy(src_hbm.at[pl.ds(cbase, NB), :], sidx)
        pltpu.sync_copy(dst_hbm.at[pl.ds(cbase, NB), :], didx)
        pltpu.async_copy(x2_hbm.at[sidx.at[0]], rows0, sem0)

        def pair_body(jj, carry):
            j = jj * 2
            pltpu.async_copy(x2_hbm.at[sidx.at[j + 1]], rows1, sem1)
            pltpu.make_async_copy(x2_hbm.at[sidx.at[j]], rows0, sem0).wait()
            pltpu.sync_copy(rows0, acc_s.at[didx.at[j]], add=True)

            @pl.when(j + 2 < NB)
            def _():
                pltpu.async_copy(x2_hbm.at[sidx.at[j + 2]], rows0, sem0)

            pltpu.make_async_copy(x2_hbm.at[sidx.at[j + 1]], rows1, sem1).wait()
            pltpu.sync_copy(rows1, acc_s.at[didx.at[j + 1]], add=True)
            return carry

        lax.fori_loop(0, NB // 2, pair_body, 0)
    plsc.subcore_barrier()

    for t in range(ROWS_S // ZROWS):
        r0 = base_row + t * ZROWS
        pltpu.sync_copy(acc_s.at[pl.ds(r0, ZROWS), :], zbuf)
        pltpu.sync_copy(zbuf, out_hbm.at[c, pl.ds(r0, ZROWS), :])


def _out_body(hist_ref, a0_ref, a1_ref, x2_ref, w_ref, b_ref, a_ref, o_ref):
    deg = (hist_ref[..., 0] + hist_ref[..., 1] + 1).astype(jnp.float32)
    dinv = lax.rsqrt(deg)
    agg = (a0_ref[...] + a1_ref[...] + x2_ref[...]) * dinv[:, None]
    h = jnp.dot(agg, w_ref[...], preferred_element_type=jnp.float32)
    h = h + b_ref[...]
    o_ref[...] = jnp.where(h > 0, h, a_ref[...] * h)


_B4 = 512


def _finalize(hist_t, acc0, acc1, x2, W, b2, a2):
    return pl.pallas_call(
        _out_body,
        grid=(N_PAD // _B4,),
        in_specs=[
            pl.BlockSpec((_B4, 2), lambda i: (i, 0)),
            pl.BlockSpec((_B4, D), lambda i: (i, 0)),
            pl.BlockSpec((_B4, D), lambda i: (i, 0)),
            pl.BlockSpec((_B4, D), lambda i: (i, 0)),
            pl.BlockSpec((D, D), lambda i: (0, 0)),
            pl.BlockSpec((1, D), lambda i: (0, 0)),
            pl.BlockSpec((1, D), lambda i: (0, 0)),
        ],
        out_specs=pl.BlockSpec((_B4, D), lambda i: (i, 0)),
        out_shape=jax.ShapeDtypeStruct((N_PAD, D), jnp.float32),
    )(hist_t, acc0, acc1, x2, W, b2, a2)


def kernel(x, edge_index, edge_type, W, b, a):
    del edge_type
    src = edge_index[0].astype(jnp.int32)
    dst = edge_index[1].astype(jnp.int32)
    src_p = jnp.concatenate([src, jnp.full((E_PAD - E,), N, jnp.int32)])
    dst_p = jnp.concatenate([dst, jnp.full((E_PAD - E,), N_PAD - 1, jnp.int32)])
    src2d = src_p.reshape(NW * NCHUNK, CHUNK)
    dst2d = dst_p.reshape(NW * NCHUNK, CHUNK)
    x_pad = jnp.zeros((N_PAD, D), jnp.float32).at[:N].set(x)

    hist = _deg_kernel(dst_p)
    hist_t = hist.T
    x2 = _prescale(hist_t, x_pad)
    acc = _agg_kernel(x2, src2d, dst2d)
    out = _finalize(hist_t, acc[0], acc[1], x2, W,
                    b.reshape(1, D), a.reshape(1, D))
    return out[:N]
